# Initial kernel scaffold; baseline (speedup 1.0000x reference)
import jax
import jax.numpy as jnp
from jax.experimental import pallas as pl


def kernel(x_stroke, edge_intersects, edge_temp_previous, edge_represented_by, edge_brepcoplanar, edge_strokecoplanar, sketch_strokes_id, W_intersects, W_temp_previous, W_represented_by, W_brepcoplanar, W_strokecoplanar, local_head_w, local_head_b, dec_w1, dec_b1, dec_w2, dec_b2):
    raise NotImplementedError("write your pallas kernel here")


# speedup vs baseline: 1.0538x; 1.0538x over previous
import jax
import jax.numpy as jnp
from jax.experimental import pallas as pl


def _head(xs_ref, agg_ref, lw_ref, lb_ref, w1_ref, b1_ref, w2_ref, b2_ref, out_ref):
    h = jax.nn.relu(xs_ref[...] + agg_ref[...])
    feat = h @ lw_ref[...] + lb_ref[...][None, :]
    z = jax.nn.relu(feat @ w1_ref[...] + b1_ref[...][None, :])
    out_ref[...] = jax.nn.sigmoid(z @ w2_ref[...] + b2_ref[...][None, :])


def kernel(x_stroke, edge_intersects, edge_temp_previous, edge_represented_by,
           edge_brepcoplanar, edge_strokecoplanar, sketch_strokes_id,
           W_intersects, W_temp_previous, W_represented_by, W_brepcoplanar,
           W_strokecoplanar, local_head_w, local_head_b, dec_w1, dec_b1,
           dec_w2, dec_b2):
    n = x_stroke.shape[0]
    sid = sketch_strokes_id.astype(jnp.float32)
    xs = x_stroke + x_stroke * sid

    def agg_mean(ei, W):
        m = (xs @ W)[ei[0]]
        s = jax.ops.segment_sum(m, ei[1], num_segments=n)
        cnt = jax.ops.segment_sum(jnp.ones((ei.shape[1],), jnp.float32), ei[1], num_segments=n)
        return s / jnp.clip(cnt, 1.0)[:, None]

    def agg_add(ei, W):
        return jax.ops.segment_sum((xs @ W)[ei[0]], ei[1], num_segments=n)

    def agg_max(ei, W):
        m = jax.ops.segment_max((xs @ W)[ei[0]], ei[1], num_segments=n)
        return jnp.where(jnp.isfinite(m), m, 0.0)

    agg = agg_mean(edge_intersects, W_intersects)
    agg = agg + agg_add(edge_temp_previous, W_temp_previous)
    agg = agg + agg_mean(edge_represented_by, W_represented_by)
    agg = agg + agg_max(edge_brepcoplanar, W_brepcoplanar)
    agg = agg + agg_max(edge_strokecoplanar, W_strokecoplanar)

    B = 4000
    out = pl.pallas_call(
        _head,
        grid=(n // B,),
        in_specs=[
            pl.BlockSpec((B, 32), lambda i: (i, 0)),
            pl.BlockSpec((B, 32), lambda i: (i, 0)),
            pl.BlockSpec((32, 64), lambda i: (0, 0)),
            pl.BlockSpec((64,), lambda i: (0,)),
            pl.BlockSpec((64, 64), lambda i: (0, 0)),
            pl.BlockSpec((64,), lambda i: (0,)),
            pl.BlockSpec((64, 1), lambda i: (0, 0)),
            pl.BlockSpec((1,), lambda i: (0,)),
        ],
        out_specs=pl.BlockSpec((B, 1), lambda i: (i, 0)),
        out_shape=jax.ShapeDtypeStruct((n, 1), jnp.float32),
    )(xs, agg, local_head_w, local_head_b, dec_w1, dec_b1, dec_w2, dec_b2)
    return out


# speedup vs baseline: 1.3510x; 1.2820x over previous
import functools
import jax
import jax.numpy as jnp
from jax import lax
from jax.experimental import pallas as pl
from jax.experimental.pallas import tpu as pltpu
from jax.experimental.pallas import tpu_sc as plsc

N = 100000
D = 32
E = 1600000
NTILES = 32
ROWS = N // NTILES
CPAD = 3136
WIN = 6400
NVEC = WIN // 16
NWIN = E // WIN
STAGE = 128
FLUSH_AT = STAGE - 16


def _splat_i32(x):
    return jnp.full((16,), x, dtype=jnp.int32)


def _make_sc_reduce(mode):
    want_cnt = mode == 'mean'
    is_max = mode == 'max'
    init_val = float('-inf') if is_max else 0.0

    out_type = [jax.ShapeDtypeStruct((N * D,), jnp.float32)]
    if want_cnt:
        out_type.append(jax.ShapeDtypeStruct((NTILES * CPAD,), jnp.float32))

    scratch = [
        pltpu.VMEM((WIN,), jnp.int32),
        pltpu.VMEM((WIN,), jnp.int32),
        pltpu.VMEM((STAGE,), jnp.int32),
        pltpu.VMEM((STAGE,), jnp.int32),
        pltpu.VMEM((STAGE, D), jnp.float32),
        pltpu.VMEM((ROWS * D,), jnp.float32),
        pltpu.VMEM((CPAD,), jnp.float32),
        pltpu.VMEM((16,), jnp.int32),
        pltpu.SemaphoreType.DMA,
    ]

    mesh = plsc.VectorSubcoreMesh(core_axis_name="c", subcore_axis_name="s")

    @functools.partial(
        pl.kernel, mesh=mesh, out_type=tuple(out_type),
        scratch_types=scratch,
        compiler_params=pltpu.CompilerParams(needs_layout_passes=False,
                                             use_tc_tiling_on_sc=False))
    def sc_kernel(*refs):
        if want_cnt:
            (table, src_hbm, dst_hbm, out_hbm, cnt_hbm, src_win, dst_win,
             idx_st, dloc_st, rows_buf, acc, cnt, cur, sem) = refs
        else:
            (table, src_hbm, dst_hbm, out_hbm, src_win, dst_win,
             idx_st, dloc_st, rows_buf, acc, cnt, cur, sem) = refs

        wid = lax.axis_index("s") * 2 + lax.axis_index("c")
        base = wid * ROWS

        for k in range(STAGE // 16):
            idx_st[pl.ds(k * 16, 16)] = (_splat_i32(wid * 128 + k * 16)
                                         + lax.iota(jnp.int32, 16))
            dloc_st[pl.ds(k * 16, 16)] = jnp.zeros((16,), jnp.int32)

        def init_acc(i, _):
            acc[pl.ds(i * 16, 16)] = jnp.full((16,), init_val, jnp.float32)
            return 0
        lax.fori_loop(0, ROWS * D // 16, init_acc, 0)

        def init_cnt(i, _):
            cnt[pl.ds(i * 16, 16)] = jnp.zeros((16,), jnp.float32)
            return 0
        if want_cnt:
            lax.fori_loop(0, CPAD // 16, init_cnt, 0)

        cur[pl.ds(0, 16)] = jnp.zeros((16,), jnp.int32)

        def process_edge(j, _):
            iota = lax.iota(jnp.int32, 16)
            jv = _splat_i32(j)
            valid = jv < cur[pl.ds(0, 16)]
            dspl = plsc.load_gather(dloc_st, [jv])
            abase = dspl * D
            rlo = plsc.load_gather(rows_buf, [jv, iota])
            rhi = plsc.load_gather(rows_buf, [jv, iota + 16])
            alo = plsc.load_gather(acc, [abase + iota])
            ahi = plsc.load_gather(acc, [abase + 16 + iota])
            if is_max:
                nlo = jnp.maximum(alo, rlo)
                nhi = jnp.maximum(ahi, rhi)
            else:
                nlo = alo + rlo
                nhi = ahi + rhi
            plsc.store_scatter(acc, [abase + iota], nlo, mask=valid)
            plsc.store_scatter(acc, [abase + 16 + iota], nhi, mask=valid)
            if want_cnt:
                lane0 = valid & (iota < _splat_i32(1))
                cc = plsc.load_gather(cnt, [dspl])
                plsc.store_scatter(cnt, [dspl], cc + 1.0, mask=lane0)
            return 0

        def flush():
            pltpu.async_copy(table.at[idx_st], rows_buf, sem).wait()
            lax.fori_loop(0, STAGE, process_edge, 0)
            cur[pl.ds(0, 16)] = jnp.zeros((16,), jnp.int32)

        def scan_vec(v, _):
            s = src_win[pl.ds(v * 16, 16)]
            d = dst_win[pl.ds(v * 16, 16)]
            dl = d - _splat_i32(base)
            m = (dl | (_splat_i32(ROWS - 1) - dl)) >= _splat_i32(0)
            mi = jnp.where(m, _splat_i32(1), _splat_i32(0))
            cums = jnp.cumsum(mi)
            cv = cur[pl.ds(0, 16)]
            pos = cv + cums - _splat_i32(1)
            plsc.store_scatter(idx_st, [pos], s, mask=m)
            plsc.store_scatter(dloc_st, [pos], dl, mask=m)
            total = cums.at[_splat_i32(15)].get(mode="promise_in_bounds")
            cv = cv + total
            cur[pl.ds(0, 16)] = cv
            lax.cond(jnp.any(cv >= _splat_i32(FLUSH_AT)), flush, lambda: None)
            return 0

        def window(w, _):
            pltpu.sync_copy(src_hbm.at[pl.ds(w * WIN, WIN)], src_win)
            pltpu.sync_copy(dst_hbm.at[pl.ds(w * WIN, WIN)], dst_win)
            lax.fori_loop(0, NVEC, scan_vec, 0)
            return 0

        lax.fori_loop(0, NWIN, window, 0)
        lax.cond(jnp.any(cur[pl.ds(0, 16)] > _splat_i32(0)),
                 flush, lambda: None)

        pltpu.sync_copy(acc, out_hbm.at[pl.ds(base * D, ROWS * D)])
        if want_cnt:
            pltpu.sync_copy(cnt, cnt_hbm.at[pl.ds(wid * CPAD, CPAD)])

    return sc_kernel


_sc_mean = _make_sc_reduce('mean')
_sc_add = _make_sc_reduce('add')
_sc_max = _make_sc_reduce('max')


def _pre_body(x_ref, sid_ref, wbc_ref, wsc_ref, xs_ref, ybc_ref, ysc_ref):
    xs = x_ref[...] * (1.0 + sid_ref[...])
    xs_ref[...] = xs
    ybc_ref[...] = xs @ wbc_ref[...]
    ysc_ref[...] = xs @ wsc_ref[...]


def _head_body(xs_ref, si_ref, ci_ref, st_ref, sr_ref, cr_ref, mb_ref, ms_ref,
               wi_ref, wt_ref, wr_ref, lw_ref, lb_ref, w1_ref, b1_ref,
               w2_ref, b2_ref, out_ref):
    mb = mb_ref[...]
    ms = ms_ref[...]
    h = xs_ref[...]
    h = h + (si_ref[...] / jnp.maximum(ci_ref[...], 1.0)) @ wi_ref[...]
    h = h + st_ref[...] @ wt_ref[...]
    h = h + (sr_ref[...] / jnp.maximum(cr_ref[...], 1.0)) @ wr_ref[...]
    h = h + jnp.where(jnp.isfinite(mb), mb, 0.0)
    h = h + jnp.where(jnp.isfinite(ms), ms, 0.0)
    h = jax.nn.relu(h)
    feat = h @ lw_ref[...] + lb_ref[...][None, :]
    z = jax.nn.relu(feat @ w1_ref[...] + b1_ref[...][None, :])
    out_ref[...] = jax.nn.sigmoid(z @ w2_ref[...] + b2_ref[...][None, :])


def kernel(x_stroke, edge_intersects, edge_temp_previous, edge_represented_by,
           edge_brepcoplanar, edge_strokecoplanar, sketch_strokes_id,
           W_intersects, W_temp_previous, W_represented_by, W_brepcoplanar,
           W_strokecoplanar, local_head_w, local_head_b, dec_w1, dec_b1,
           dec_w2, dec_b2):
    sid = sketch_strokes_id.astype(jnp.float32)
    B = 4000
    nb = N // B

    xs, ybc, ysc = pl.pallas_call(
        _pre_body,
        grid=(nb,),
        in_specs=[
            pl.BlockSpec((B, D), lambda i: (i, 0)),
            pl.BlockSpec((B, 1), lambda i: (i, 0)),
            pl.BlockSpec((D, D), lambda i: (0, 0)),
            pl.BlockSpec((D, D), lambda i: (0, 0)),
        ],
        out_specs=[
            pl.BlockSpec((B, D), lambda i: (i, 0)),
            pl.BlockSpec((B, D), lambda i: (i, 0)),
            pl.BlockSpec((B, D), lambda i: (i, 0)),
        ],
        out_shape=[
            jax.ShapeDtypeStruct((N, D), jnp.float32),
            jax.ShapeDtypeStruct((N, D), jnp.float32),
            jax.ShapeDtypeStruct((N, D), jnp.float32),
        ],
    )(x_stroke, sid, W_brepcoplanar, W_strokecoplanar)

    def unpack_cnt(c):
        return c.reshape(NTILES, CPAD)[:, :ROWS].reshape(N, 1)

    s_int, c_int = _sc_mean(xs, edge_intersects[0], edge_intersects[1])
    s_tp, = _sc_add(xs, edge_temp_previous[0], edge_temp_previous[1])
    s_rb, c_rb = _sc_mean(xs, edge_represented_by[0], edge_represented_by[1])
    m_bc, = _sc_max(ybc, edge_brepcoplanar[0], edge_brepcoplanar[1])
    m_sc, = _sc_max(ysc, edge_strokecoplanar[0], edge_strokecoplanar[1])

    out = pl.pallas_call(
        _head_body,
        grid=(nb,),
        in_specs=[
            pl.BlockSpec((B, D), lambda i: (i, 0)),
            pl.BlockSpec((B, D), lambda i: (i, 0)),
            pl.BlockSpec((B, 1), lambda i: (i, 0)),
            pl.BlockSpec((B, D), lambda i: (i, 0)),
            pl.BlockSpec((B, D), lambda i: (i, 0)),
            pl.BlockSpec((B, 1), lambda i: (i, 0)),
            pl.BlockSpec((B, D), lambda i: (i, 0)),
            pl.BlockSpec((B, D), lambda i: (i, 0)),
            pl.BlockSpec((D, D), lambda i: (0, 0)),
            pl.BlockSpec((D, D), lambda i: (0, 0)),
            pl.BlockSpec((D, D), lambda i: (0, 0)),
            pl.BlockSpec((D, 64), lambda i: (0, 0)),
            pl.BlockSpec((64,), lambda i: (0,)),
            pl.BlockSpec((64, 64), lambda i: (0, 0)),
            pl.BlockSpec((64,), lambda i: (0,)),
            pl.BlockSpec((64, 1), lambda i: (0, 0)),
            pl.BlockSpec((1,), lambda i: (0,)),
        ],
        out_specs=pl.BlockSpec((B, 1), lambda i: (i, 0)),
        out_shape=jax.ShapeDtypeStruct((N, 1), jnp.float32),
    )(xs, s_int.reshape(N, D), unpack_cnt(c_int), s_tp.reshape(N, D),
      s_rb.reshape(N, D), unpack_cnt(c_rb), m_bc.reshape(N, D),
      m_sc.reshape(N, D), W_intersects, W_temp_previous, W_represented_by,
      local_head_w, local_head_b, dec_w1, dec_b1, dec_w2, dec_b2)
    return out
